# Initial kernel scaffold; baseline (speedup 1.0000x reference)
import jax
import jax.numpy as jnp
from jax.experimental import pallas as pl


def kernel(input_image, V, x_data, y_data):
    raise NotImplementedError("write your pallas kernel here")


# speedup vs baseline: 1.7448x; 1.7448x over previous
import functools

import jax
import jax.numpy as jnp
from jax import lax
from jax.experimental import pallas as pl
from jax.experimental.pallas import tpu as pltpu
from jax.experimental.pallas import tpu_sc as plsc

L = 16


def _sc_pipeline(HW, n, B, n_workers, chunk):
    groups = chunk // L
    n_chunks = HW // (n_workers * chunk)
    pw = HW // n_workers

    mesh = plsc.VectorSubcoreMesh(core_axis_name="c", subcore_axis_name="s")
    info = plsc.get_sparse_core_info()
    nc = info.num_cores

    @functools.partial(
        pl.kernel,
        out_type=jax.ShapeDtypeStruct((B, 3, HW), jnp.float32),
        mesh=mesh,
        compiler_params=pltpu.CompilerParams(
            needs_layout_passes=False, use_tc_tiling_on_sc=False
        ),
        scratch_types=[
            pltpu.VMEM((3, chunk, n), jnp.float32),
            pltpu.VMEM((3, chunk, n), jnp.float32),
            pltpu.VMEM((B, 3, chunk), jnp.float32),
            pltpu.VMEM((chunk, 9), jnp.float32),
            pltpu.VMEM((B, 3, chunk), jnp.float32),
        ],
    )
    def run(in_hbm, x_hbm, y_hbm, v_hbm, out_hbm, x_v, y_v, in_v, v_v, out_v):
        wid = lax.axis_index("s") * nc + lax.axis_index("c")
        base = wid * pw

        @pl.loop(0, n_chunks)
        def _chunk(ci):
            p0 = base + ci * chunk
            pltpu.sync_copy(x_hbm.at[:, pl.ds(p0, chunk), :], x_v)
            pltpu.sync_copy(y_hbm.at[:, pl.ds(p0, chunk), :], y_v)
            pltpu.sync_copy(in_hbm.at[:, :, pl.ds(p0, chunk)], in_v)
            pltpu.sync_copy(v_hbm.at[pl.ds(p0, chunk), :], v_v)

            @pl.loop(0, groups)
            def _group(g):
                rows = g * L + lax.iota(jnp.int32, L)
                after = [[None] * 3 for _ in range(B)]
                for c in range(3):
                    cvec = jnp.full((L,), c, jnp.int32)
                    for b in range(B):
                        xi = in_v[b, c, pl.ds(g * L, L)]
                        lo = jnp.zeros((L,), jnp.int32)
                        for s in (8, 4, 2, 1):
                            m = lo + s
                            xm = plsc.load_gather(x_v, [cvec, rows, m - 1])
                            lo = jnp.where(xm < xi, m, lo)
                        idx = jnp.maximum(lo, 1)
                        x0 = plsc.load_gather(x_v, [cvec, rows, idx - 1])
                        x1 = plsc.load_gather(x_v, [cvec, rows, idx])
                        y0 = plsc.load_gather(y_v, [cvec, rows, idx - 1])
                        y1 = plsc.load_gather(y_v, [cvec, rows, idx])
                        alpha = (xi - x0) / (x1 - x0 + 1e-8)
                        after[b][c] = y0 + alpha * (y1 - y0)
                for d in range(3):
                    acc = [jnp.zeros((L,), jnp.float32) for _ in range(B)]
                    for c in range(3):
                        vcd = plsc.load_gather(
                            v_v, [rows, jnp.full((L,), c * 3 + d, jnp.int32)]
                        )
                        for b in range(B):
                            acc[b] = acc[b] + after[b][c] * vcd
                    for b in range(B):
                        out_v[b, d, pl.ds(g * L, L)] = jnp.clip(acc[b], 0.0, 1.0)

            pltpu.sync_copy(out_v, out_hbm.at[:, :, pl.ds(p0, chunk)])

    return run


@jax.jit
def kernel(input_image, V, x_data, y_data):
    B, _, H, W = input_image.shape
    n = x_data.shape[-1]
    HW = H * W
    in2 = input_image.reshape(B, 3, HW)
    x2 = x_data.reshape(3, HW, n)
    y2 = y_data.reshape(3, HW, n)
    v2 = V.reshape(HW, 9)
    out = _sc_pipeline(HW, n, B, n_workers=32, chunk=256)(in2, x2, y2, v2)
    return out.reshape(B, 3, H, W)
